# baseline (speedup 1.0000x reference)
import functools

import jax
import jax.numpy as jnp
from jax import lax
from jax.experimental import pallas as pl
from jax.experimental.pallas import tpu as pltpu
from jax.experimental.pallas import tpu_sc as plsc

NA = 10000
NB = 320000
NEI = 32
D = 128
BD = 16
DEPTH_M1 = 3

NC = 2
NS = 16
NW = NC * NS

NAP = 10240

F32 = jnp.float32


@functools.lru_cache(maxsize=None)
def _make_segsum(n_seg):
    groups = D // 16
    CH = 8
    RCH = CH * NEI
    SPLIT = 2
    SR = RCH // SPLIT
    per_w = NAP // NW
    n_chunks = per_w // CH
    NBUF = 2

    mesh = plsc.VectorSubcoreMesh(core_axis_name="c", subcore_axis_name="s")

    @functools.partial(
        pl.kernel, mesh=mesh,
        out_type=tuple(jax.ShapeDtypeStruct((NAP, D), F32)
                       for _ in range(n_seg)),
        scratch_types=[
            pltpu.VMEM((per_w * NEI,), jnp.int32),
            [pltpu.VMEM((RCH, D), F32) for _ in range(NBUF)],
            [pltpu.VMEM((CH, D), F32) for _ in range(NBUF)],
            [pltpu.SemaphoreType.DMA for _ in range(NBUF)],
            [pltpu.SemaphoreType.DMA for _ in range(NBUF)],
        ],
    )
    def seg_kernel(*refs):
        tables = refs[0:2 * n_seg:2]
        idxs = refs[1:2 * n_seg:2]
        outs = refs[2 * n_seg:3 * n_seg]
        idx_v, bufs, ovs, sems, sow = refs[3 * n_seg:]
        w = lax.axis_index("s") * NC + lax.axis_index("c")

        def run_segment(table, idx, out):
            base = pl.multiple_of(w * (per_w * NEI), per_w * NEI)
            obase = w * per_w
            pltpu.sync_copy(idx.at[pl.ds(base, per_w * NEI)], idx_v)

            def fire(c, b):
                off = pl.multiple_of(
                    jnp.minimum(c, n_chunks - 1) * RCH, RCH)
                for q in range(SPLIT):
                    pltpu.async_copy(
                        table.at[idx_v.at[pl.ds(off + q * SR, SR)]],
                        bufs[b].at[pl.ds(q * SR, SR)], sems[b])

            def wait(b):
                for q in range(SPLIT):
                    pltpu.make_async_copy(
                        table.at[idx_v.at[pl.ds(0, SR)]],
                        bufs[b].at[pl.ds(q * SR, SR)], sems[b]).wait()

            def compute(b):
                for a in range(CH):
                    accs = tuple(bufs[b][a * NEI, pl.ds(16 * k, 16)]
                                 for k in range(groups))

                    def body(j, accs, _a=a, _b=b):
                        return tuple(
                            accs[k]
                            + bufs[_b][_a * NEI + j, pl.ds(16 * k, 16)]
                            for k in range(groups))

                    accs = lax.fori_loop(1, NEI, body, accs, unroll=4)
                    for k in range(groups):
                        ovs[b][a, pl.ds(16 * k, 16)] = accs[k]

            def fire_out(c, b):
                pltpu.async_copy(ovs[b], out.at[pl.ds(obase + c * CH, CH)],
                                 sow[b])

            def wait_out(b):
                pltpu.make_async_copy(ovs[b], out.at[pl.ds(0, CH)],
                                      sow[b]).wait()

            for b in range(NBUF):
                fire(b, b)

            for b in range(NBUF):
                wait(b)
                compute(b)
                fire_out(b, b)
                fire(b + NBUF, b)

            @pl.loop(1, n_chunks // NBUF)
            def _grp(gidx):
                for b in range(NBUF):
                    c = gidx * NBUF + b
                    wait(b)
                    wait_out(b)
                    compute(b)
                    fire_out(c, b)
                    fire(c + NBUF, b)

            for b in range(NBUF):
                wait(b)
                wait_out(b)

        for s in range(n_seg):
            run_segment(tables[s], idxs[s], outs[s])

    return seg_kernel


G3CH = 120
NBP = 322560


@functools.lru_cache(maxsize=None)
def _make_gather3():
    CH = G3CH
    USPLIT = 3
    UR = CH // USPLIT
    per_w = NBP // NW
    n_chunks = per_w // CH
    NBUF = 2

    mesh = plsc.VectorSubcoreMesh(core_axis_name="c", subcore_axis_name="s")

    @functools.partial(
        pl.kernel, mesh=mesh,
        out_type=jax.ShapeDtypeStruct((NBP, D), F32),
        scratch_types=[
            [pltpu.VMEM((CH,), jnp.int32) for _ in range(NBUF)],
            [pltpu.VMEM((CH,), jnp.int32) for _ in range(NBUF)],
            [pltpu.VMEM((CH,), jnp.int32) for _ in range(NBUF)],
            [pltpu.VMEM((CH, D), F32) for _ in range(NBUF)],
            [pltpu.VMEM((CH, D), F32) for _ in range(NBUF)],
            [pltpu.VMEM((CH, D), F32) for _ in range(NBUF)],
            [pltpu.VMEM((CH, D), F32) for _ in range(NBUF)],
            [pltpu.SemaphoreType.DMA for _ in range(NBUF)],
            [pltpu.SemaphoreType.DMA for _ in range(NBUF)],
            [pltpu.SemaphoreType.DMA for _ in range(NBUF)],
        ],
    )
    def gather3_kernel(t, vp, u, ia, ir, ib, g,
                       iav, irv, ibv, tv, vv, uv, gv, si, sg, sw):
        w = lax.axis_index("s") * NC + lax.axis_index("c")
        base = w * per_w
        last = n_chunks - 1

        def fire_idx(c, b):
            off = pl.multiple_of(base + jnp.minimum(c, last) * CH, 8)
            pltpu.async_copy(ia.at[pl.ds(off, CH)], iav[b], si[b])
            pltpu.async_copy(ir.at[pl.ds(off, CH)], irv[b], si[b])
            pltpu.async_copy(ib.at[pl.ds(off, CH)], ibv[b], si[b])

        def wait_idx(b):
            pltpu.make_async_copy(ia.at[pl.ds(0, CH)], iav[b], si[b]).wait()
            pltpu.make_async_copy(ir.at[pl.ds(0, CH)], irv[b], si[b]).wait()
            pltpu.make_async_copy(ib.at[pl.ds(0, CH)], ibv[b], si[b]).wait()

        def fire_gather(b):
            pltpu.async_copy(t.at[iav[b]], tv[b], sg[b])
            pltpu.async_copy(vp.at[irv[b]], vv[b], sg[b])
            for q in range(USPLIT):
                pltpu.async_copy(u.at[ibv[b].at[pl.ds(q * UR, UR)]],
                                 uv[b].at[pl.ds(q * UR, UR)], sg[b])

        def wait_gather(b):
            pltpu.make_async_copy(t.at[iav[b]], tv[b], sg[b]).wait()
            pltpu.make_async_copy(vp.at[irv[b]], vv[b], sg[b]).wait()
            for q in range(USPLIT):
                pltpu.make_async_copy(u.at[ibv[b].at[pl.ds(0, UR)]],
                                      uv[b].at[pl.ds(q * UR, UR)],
                                      sg[b]).wait()

        def fire_write(c, b):
            off = pl.multiple_of(base + c * CH, 8)
            pltpu.async_copy(gv[b], g.at[pl.ds(off, CH)], sw[b])

        def wait_write(b):
            pltpu.make_async_copy(gv[b], g.at[pl.ds(0, CH)], sw[b]).wait()

        def compute(b):
            @pl.loop(0, CH)
            def _row(r):
                for k in range(8):
                    sl = pl.ds(16 * k, 16)
                    gv[b][r, sl] = tv[b][r, sl] - vv[b][r, sl] - uv[b][r, sl]

        for b in range(NBUF):
            fire_idx(b, b)
        for b in range(NBUF):
            wait_idx(b)
            fire_gather(b)

        for b in range(NBUF):
            wait_gather(b)
            fire_idx(b + NBUF, b)
            compute(b)
            fire_write(b, b)
            wait_idx(b)
            fire_gather(b)

        @pl.loop(1, n_chunks // NBUF)
        def _grp(gidx):
            for b in range(NBUF):
                c = gidx * NBUF + b
                wait_gather(b)
                fire_idx(c + NBUF, b)
                wait_write(b)
                compute(b)
                fire_write(c, b)
                wait_idx(b)
                fire_gather(b)

        for b in range(NBUF):
            wait_gather(b)
            wait_write(b)

    return gather3_kernel


def _dot(a, b):
    return jnp.dot(a, b, preferred_element_type=F32)


def _row_spec(blk, d):
    return pl.BlockSpec((blk, d), lambda i: (i, 0))


def _rep_spec(shape):
    return pl.BlockSpec(shape, lambda i: (0,) * len(shape))


BA = 1024
BB = 2560


def _tc_init_atom(af, agg_af, agg_bfn, W_nin, b_nin, b_node,
                  We2, Wno0, b_nout, Weo0, b_eout):
    def body(af_r, gaf_r, gbfn_r, wnin_r, bnin_r, bnode_r, we2_r,
             wno0_r, bnout_r, weo0_r, beout_r,
             am0_r, abias_r, tconst_r, w3_r, afno_r, afeo_r):
        af_v = af_r[...]
        ai = _dot(af_v, wnin_r[...]) + bnin_r[...]
        am0_r[...] = jnp.maximum(ai, 0.0)
        abias_r[...] = ai + gbfn_r[...] + bnode_r[...]
        tconst_r[...] = _dot(gaf_r[...], we2_r[...])
        w3_r[...] = _dot(af_v, we2_r[...])
        afno_r[...] = _dot(af_v, wno0_r[...]) + bnout_r[...]
        afeo_r[...] = _dot(af_v, weo0_r[...]) + beout_r[...]

    n = NAP // BA
    f = jax.ShapeDtypeStruct((NAP, D), F32)
    return pl.pallas_call(
        body,
        grid=(n,),
        in_specs=[_row_spec(BA, D), _row_spec(BA, D), _row_spec(BA, D),
                  _rep_spec((D, D)), _rep_spec((1, D)),
                  _rep_spec((1, D)), _rep_spec((D, D)), _rep_spec((D, D)),
                  _rep_spec((1, D)), _rep_spec((D, D)), _rep_spec((1, D))],
        out_specs=[_row_spec(BA, D)] * 6,
        out_shape=[f] * 6,
    )(af, agg_af, agg_bfn, W_nin, b_nin, b_node, We2,
      Wno0, b_nout, Weo0, b_eout)


def _tc_init_bond(bf, W_ein, b_ein, b_edge, We0, Wn2):
    def body(bf_r, wein_r, bein_r, bedge_r, we0_r, wn2_r,
             bm0_r, bias2_r, u0_r, bfn_r):
        bf_v = bf_r[...]
        bi = _dot(bf_v, wein_r[...]) + bein_r[...]
        bm0 = jnp.maximum(bi, 0.0)
        bm0_r[...] = bm0
        bias2_r[...] = bi + bedge_r[...]
        u0_r[...] = _dot(bm0, we0_r[...])
        bfn_r[...] = _dot(bf_v, wn2_r[...])

    n = NB // BB
    f = jax.ShapeDtypeStruct((NB, D), F32)
    return pl.pallas_call(
        body,
        grid=(n,),
        in_specs=[_row_spec(BB, BD), _rep_spec((BD, D)), _rep_spec((1, D)),
                  _rep_spec((1, D)), _rep_spec((D, D)), _rep_spec((BD, D))],
        out_specs=[_row_spec(BB, D)] * 4,
        out_shape=[f] * 4,
    )(bf, W_ein, b_ein, b_edge, We0, Wn2)


def _tc_atom_update(abias, G1, G2, w3, Wn0, Wn1, We1):
    def body(abias_r, g1_r, g2_r, w3_r, wn0_r, wn1_r, we1_r, am_r, vp_r):
        am = jnp.maximum(
            abias_r[...] + _dot(g1_r[...], wn0_r[...])
            + _dot(g2_r[...], wn1_r[...]), 0.0)
        am_r[...] = am
        vp_r[...] = _dot(am, we1_r[...]) + w3_r[...]

    n = NAP // BA
    f = jax.ShapeDtypeStruct((NAP, D), F32)
    return pl.pallas_call(
        body,
        grid=(n,),
        in_specs=[_row_spec(BA, D)] * 4 + [_rep_spec((D, D))] * 3,
        out_specs=[_row_spec(BA, D)] * 2,
        out_shape=[f] * 2,
    )(abias, G1, G2, w3, Wn0, Wn1, We1)


def _tc_t(G2, G3, tconst, We0, We1):
    def body(g2_r, g3_r, tc_r, we0_r, we1_r, t_r):
        t_r[...] = (_dot(g2_r[...], we0_r[...]) + _dot(g3_r[...], we1_r[...])
                    + tc_r[...])

    n = NAP // BA
    return pl.pallas_call(
        body,
        grid=(n,),
        in_specs=[_row_spec(BA, D)] * 3 + [_rep_spec((D, D))] * 2,
        out_specs=_row_spec(BA, D),
        out_shape=jax.ShapeDtypeStruct((NAP, D), F32),
    )(G2, G3, tconst, We0, We1)


def _tc_bond_update(bias2, g, We0, with_u):
    def body_u(bias2_r, g_r, we0_r, bm_r, u_r):
        bm = jnp.maximum(bias2_r[...] + g_r[...], 0.0)
        bm_r[...] = bm
        u_r[...] = _dot(bm, we0_r[...])

    def body_nou(bias2_r, g_r, we0_r, bm_r):
        bm_r[...] = jnp.maximum(bias2_r[...] + g_r[...], 0.0)

    n = NB // BB
    f = jax.ShapeDtypeStruct((NB, D), F32)
    if with_u:
        return pl.pallas_call(
            body_u,
            grid=(n,),
            in_specs=[_row_spec(BB, D), _row_spec(BB, D), _rep_spec((D, D))],
            out_specs=[_row_spec(BB, D)] * 2,
            out_shape=[f] * 2,
        )(bias2, g, We0)
    return pl.pallas_call(
        body_nou,
        grid=(n,),
        in_specs=[_row_spec(BB, D), _row_spec(BB, D), _rep_spec((D, D))],
        out_specs=_row_spec(BB, D),
        out_shape=f,
    )(bias2, g, We0)


def _tc_final(afno, A, Wno1, afeo, B, Weo1):
    def body(afno_r, a_r, wno1_r, afeo_r, b_r, weo1_r, ao_r, bo_r):
        ao_r[...] = jnp.maximum(afno_r[...] + _dot(a_r[...], wno1_r[...]), 0.0)
        bo_r[...] = jnp.maximum(afeo_r[...] + _dot(b_r[...], weo1_r[...]), 0.0)

    n = NAP // BA
    f = jax.ShapeDtypeStruct((NAP, D), F32)
    return pl.pallas_call(
        body,
        grid=(n,),
        in_specs=[_row_spec(BA, D), _row_spec(BA, D), _rep_spec((D, D)),
                  _row_spec(BA, D), _row_spec(BA, D), _rep_spec((D, D))],
        out_specs=[_row_spec(BA, D)] * 2,
        out_shape=[f] * 2,
    )(afno, A, Wno1, afeo, B, Weo1)


def kernel(atom_features, bond_features, a2a, a2b, b2a, b2revb,
           W_nin, b_nin, W_ein, b_ein, W_node, b_node, W_edge, b_edge,
           W_nout, b_nout, W_eout, b_eout):
    H = D
    Wn0, Wn1, Wn2 = W_node[:H], W_node[H:2 * H], W_node[2 * H:]
    We0, We1, We2 = W_edge[:H], W_edge[H:2 * H], W_edge[2 * H:]
    Wno0, Wno1 = W_nout[:H], W_nout[H:]
    Weo0, Weo1 = W_eout[:H], W_eout[H:]
    b_nin2 = b_nin[None, :]
    b_ein2 = b_ein[None, :]
    b_node2 = b_node[None, :]
    b_edge2 = b_edge[None, :]
    b_nout2 = b_nout[None, :]
    b_eout2 = b_eout[None, :]

    af_p = jnp.pad(atom_features, ((0, NAP - NA), (0, 0)))
    a2a_f = jnp.pad(a2a, ((0, NAP - NA), (0, 0))).reshape(-1)
    a2b_f = jnp.pad(a2b, ((0, NAP - NA), (0, 0))).reshape(-1)
    rev_a = jnp.take(b2a, b2revb)
    b2a_p = jnp.pad(b2a, (0, NBP - NB))
    rev_a_p = jnp.pad(rev_a, (0, NBP - NB))
    b2revb_p = jnp.pad(b2revb, (0, NBP - NB))

    seg1 = _make_segsum(1)
    seg2 = _make_segsum(2)
    gather3 = _make_gather3()

    bm, bias2, u, bfn = _tc_init_bond(
        bond_features, W_ein, b_ein2, b_edge2, We0, Wn2)
    agg_af, agg_bfn = seg2(af_p, a2a_f, bfn, a2b_f)
    am, abias, tconst, w3, afno, afeo = _tc_init_atom(
        af_p, agg_af, agg_bfn, W_nin, b_nin2, b_node2,
        We2, Wno0, b_nout2, Weo0, b_eout2)

    G3 = None
    for step in range(DEPTH_M1):
        G1, G2 = seg2(am, a2a_f, bm, a2b_f)
        am, vp = _tc_atom_update(abias, G1, G2, w3, Wn0, Wn1, We1)
        (G3,) = seg1(am, a2a_f)
        t = _tc_t(G2, G3, tconst, We0, We1)
        g = gather3(t, vp, u, b2a_p, rev_a_p, b2revb_p)
        last = step == DEPTH_M1 - 1
        if last:
            bm = _tc_bond_update(bias2, g, We0, with_u=False)
        else:
            bm, u = _tc_bond_update(bias2, g, We0, with_u=True)

    (B,) = seg1(bm, a2b_f)
    atom_out, bond_out = _tc_final(afno, G3, Wno1, afeo, B, Weo1)
    return atom_out[:NA], bond_out[:NA]


# speedup vs baseline: 1.5568x; 1.5568x over previous
import functools

import jax
import jax.numpy as jnp
from jax import lax
from jax.experimental import pallas as pl
from jax.experimental.pallas import tpu as pltpu
from jax.experimental.pallas import tpu_sc as plsc

NA = 10000
NB = 320000
NEI = 32
D = 128
BD = 16
DEPTH_M1 = 3

NC = 2
NS = 16
NW = NC * NS

NAP = 10240

F32 = jnp.float32


@functools.lru_cache(maxsize=None)
def _make_segsum_small():
    groups = D // 16
    CH = 4
    RCH = CH * NEI
    per_w = NAP // NW
    n_chunks = per_w // CH
    n_super = n_chunks // 4

    mesh = plsc.VectorSubcoreMesh(core_axis_name="c", subcore_axis_name="s")

    @functools.partial(
        pl.kernel, mesh=mesh,
        out_type=jax.ShapeDtypeStruct((NAP, D), F32),
        scratch_types=[
            [pltpu.VMEM((RCH,), jnp.int32) for _ in range(2)],
            [pltpu.VMEM((RCH, D), F32) for _ in range(2)],
            [pltpu.VMEM((2 * CH, D), F32) for _ in range(2)],
            [pltpu.SemaphoreType.DMA for _ in range(2)],
            [pltpu.SemaphoreType.DMA for _ in range(2)],
            [pltpu.SemaphoreType.DMA for _ in range(2)],
            pltpu.VMEM_SHARED((NAP, D), F32),
        ],
    )
    def seg_kernel(table, idx, out, idxv, bufs, ovs, si, sg, so, spm):
        sid = lax.axis_index("s")
        w = sid * NC + lax.axis_index("c")
        base = pl.multiple_of(w * (per_w * NEI), per_w * NEI)
        obase = w * per_w
        rows = NAP // NS
        soff = pl.multiple_of(sid * rows, rows)
        pltpu.sync_copy(table.at[pl.ds(soff, rows)],
                        spm.at[pl.ds(soff, rows)])
        plsc.subcore_barrier()
        last = n_chunks - 1

        def fire_idx(c, b):
            off = pl.multiple_of(base + jnp.minimum(c, last) * RCH, RCH)
            pltpu.async_copy(idx.at[pl.ds(off, RCH)], idxv[b], si[b])

        def wait_idx(b):
            pltpu.make_async_copy(idx.at[pl.ds(0, RCH)], idxv[b],
                                  si[b]).wait()

        def fire_gather(b):
            pltpu.async_copy(spm.at[idxv[b]], bufs[b], sg[b])

        def wait_gather(b):
            pltpu.make_async_copy(spm.at[idxv[b]], bufs[b], sg[b]).wait()

        def fire_out(p, pp):
            pltpu.async_copy(ovs[pp],
                             out.at[pl.ds(obase + p * 2 * CH, 2 * CH)],
                             so[pp])

        def wait_out(pp):
            pltpu.make_async_copy(ovs[pp], out.at[pl.ds(0, 2 * CH)],
                                  so[pp]).wait()

        def compute(b, pp):
            for a in range(CH):
                accs = tuple(bufs[b][a * NEI, pl.ds(16 * k, 16)]
                             for k in range(groups))

                def body(j, accs, _a=a, _b=b):
                    return tuple(
                        accs[k] + bufs[_b][_a * NEI + j, pl.ds(16 * k, 16)]
                        for k in range(groups))

                accs = lax.fori_loop(1, NEI, body, accs, unroll=4)
                for k in range(groups):
                    ovs[pp][b * CH + a, pl.ds(16 * k, 16)] = accs[k]

        for b in range(2):
            fire_idx(b, b)
            wait_idx(b)
            fire_gather(b)

        def do_super(g, peeled):
            for pp in range(2):
                p = g * 2 + pp
                if not peeled:
                    wait_out(pp)
                for b in range(2):
                    c = p * 2 + b
                    wait_gather(b)
                    fire_idx(c + 2, b)
                    compute(b, pp)
                    wait_idx(b)
                    fire_gather(b)
                fire_out(p, pp)

        do_super(0, True)

        @pl.loop(1, n_super)
        def _g(g):
            do_super(g, False)

        for b in range(2):
            wait_gather(b)
            wait_out(b)

    return seg_kernel


@functools.lru_cache(maxsize=None)
def _make_segsum_big():
    groups = D // 16
    CH = 4
    RCH = CH * NEI
    per_w = NAP // NW
    n_chunks = per_w // CH
    NBUF = 4

    mesh = plsc.VectorSubcoreMesh(core_axis_name="c", subcore_axis_name="s")

    @functools.partial(
        pl.kernel, mesh=mesh,
        out_type=jax.ShapeDtypeStruct((NAP, D), F32),
        scratch_types=[
            pltpu.VMEM((per_w * NEI,), jnp.int32),
            [pltpu.VMEM((RCH, D), F32) for _ in range(NBUF)],
            pltpu.VMEM((per_w, D), F32),
            [pltpu.SemaphoreType.DMA for _ in range(NBUF)],
        ],
    )
    def seg_kernel(table, idx, out, idx_v, bufs, out_v, sems):
        w = lax.axis_index("s") * NC + lax.axis_index("c")
        base = pl.multiple_of(w * (per_w * NEI), per_w * NEI)
        pltpu.sync_copy(idx.at[pl.ds(base, per_w * NEI)], idx_v)

        def fire(c, b):
            off = pl.multiple_of(jnp.minimum(c, n_chunks - 1) * RCH, RCH)
            pltpu.async_copy(table.at[idx_v.at[pl.ds(off, RCH)]],
                             bufs[b], sems[b])

        def wait(b):
            pltpu.make_async_copy(table.at[idx_v.at[pl.ds(0, RCH)]],
                                  bufs[b], sems[b]).wait()

        for b in range(NBUF):
            fire(b, b)

        @pl.loop(0, n_chunks // NBUF)
        def _grp(gidx):
            for b in range(NBUF):
                c = gidx * NBUF + b
                wait(b)
                for a in range(CH):
                    accs = tuple(bufs[b][a * NEI, pl.ds(16 * k, 16)]
                                 for k in range(groups))

                    def body(j, accs, _a=a, _b=b):
                        return tuple(
                            accs[k] + bufs[_b][_a * NEI + j, pl.ds(16 * k, 16)]
                            for k in range(groups))

                    accs = lax.fori_loop(1, NEI, body, accs, unroll=4)
                    for k in range(groups):
                        out_v[c * CH + a, pl.ds(16 * k, 16)] = accs[k]
                fire(c + NBUF, b)

        for b in range(NBUF):
            wait(b)
        pltpu.sync_copy(
            out_v, out.at[pl.ds(pl.multiple_of(w * per_w, per_w), per_w)])

    return seg_kernel


G3CH = 120
NBP = 322560


@functools.lru_cache(maxsize=None)
def _make_gather3():
    CH = G3CH
    USPLIT = 3
    UR = CH // USPLIT
    per_w = NBP // NW
    n_chunks = per_w // CH
    NBUF = 2

    mesh = plsc.VectorSubcoreMesh(core_axis_name="c", subcore_axis_name="s")

    @functools.partial(
        pl.kernel, mesh=mesh,
        out_type=jax.ShapeDtypeStruct((NBP, D), F32),
        scratch_types=[
            [pltpu.VMEM((CH,), jnp.int32) for _ in range(NBUF)],
            [pltpu.VMEM((CH,), jnp.int32) for _ in range(NBUF)],
            [pltpu.VMEM((CH,), jnp.int32) for _ in range(NBUF)],
            [pltpu.VMEM((CH, D), F32) for _ in range(NBUF)],
            [pltpu.VMEM((CH, D), F32) for _ in range(NBUF)],
            [pltpu.VMEM((CH, D), F32) for _ in range(NBUF)],
            [pltpu.VMEM((CH, D), F32) for _ in range(NBUF)],
            [pltpu.SemaphoreType.DMA for _ in range(NBUF)],
            [pltpu.SemaphoreType.DMA for _ in range(NBUF)],
            [pltpu.SemaphoreType.DMA for _ in range(NBUF)],
        ],
    )
    def gather3_kernel(t, vp, u, ia, ir, ib, g,
                       iav, irv, ibv, tv, vv, uv, gv, si, sg, sw):
        w = lax.axis_index("s") * NC + lax.axis_index("c")
        base = w * per_w
        last = n_chunks - 1

        def fire_idx(c, b):
            off = pl.multiple_of(base + jnp.minimum(c, last) * CH, 8)
            pltpu.async_copy(ia.at[pl.ds(off, CH)], iav[b], si[b])
            pltpu.async_copy(ir.at[pl.ds(off, CH)], irv[b], si[b])
            pltpu.async_copy(ib.at[pl.ds(off, CH)], ibv[b], si[b])

        def wait_idx(b):
            pltpu.make_async_copy(ia.at[pl.ds(0, CH)], iav[b], si[b]).wait()
            pltpu.make_async_copy(ir.at[pl.ds(0, CH)], irv[b], si[b]).wait()
            pltpu.make_async_copy(ib.at[pl.ds(0, CH)], ibv[b], si[b]).wait()

        def fire_gather(b):
            pltpu.async_copy(t.at[iav[b]], tv[b], sg[b])
            pltpu.async_copy(vp.at[irv[b]], vv[b], sg[b])
            for q in range(USPLIT):
                pltpu.async_copy(u.at[ibv[b].at[pl.ds(q * UR, UR)]],
                                 uv[b].at[pl.ds(q * UR, UR)], sg[b])

        def wait_gather(b):
            pltpu.make_async_copy(t.at[iav[b]], tv[b], sg[b]).wait()
            pltpu.make_async_copy(vp.at[irv[b]], vv[b], sg[b]).wait()
            for q in range(USPLIT):
                pltpu.make_async_copy(u.at[ibv[b].at[pl.ds(0, UR)]],
                                      uv[b].at[pl.ds(q * UR, UR)],
                                      sg[b]).wait()

        def fire_write(c, b):
            off = pl.multiple_of(base + c * CH, 8)
            pltpu.async_copy(gv[b], g.at[pl.ds(off, CH)], sw[b])

        def wait_write(b):
            pltpu.make_async_copy(gv[b], g.at[pl.ds(0, CH)], sw[b]).wait()

        def compute(b):
            @pl.loop(0, CH)
            def _row(r):
                for k in range(8):
                    sl = pl.ds(16 * k, 16)
                    gv[b][r, sl] = tv[b][r, sl] - vv[b][r, sl] - uv[b][r, sl]

        for b in range(NBUF):
            fire_idx(b, b)
        for b in range(NBUF):
            wait_idx(b)
            fire_gather(b)

        for b in range(NBUF):
            wait_gather(b)
            fire_idx(b + NBUF, b)
            compute(b)
            fire_write(b, b)
            wait_idx(b)
            fire_gather(b)

        @pl.loop(1, n_chunks // NBUF)
        def _grp(gidx):
            for b in range(NBUF):
                c = gidx * NBUF + b
                wait_gather(b)
                fire_idx(c + NBUF, b)
                wait_write(b)
                compute(b)
                fire_write(c, b)
                wait_idx(b)
                fire_gather(b)

        for b in range(NBUF):
            wait_gather(b)
            wait_write(b)

    return gather3_kernel


def _dot(a, b):
    return jnp.dot(a, b, preferred_element_type=F32)


def _row_spec(blk, d):
    return pl.BlockSpec((blk, d), lambda i: (i, 0))


def _rep_spec(shape):
    return pl.BlockSpec(shape, lambda i: (0,) * len(shape))


BA = 1024
BB = 2560


def _tc_init_atom(af, agg_af, agg_bfn, W_nin, b_nin, b_node,
                  We2, Wno0, b_nout, Weo0, b_eout):
    def body(af_r, gaf_r, gbfn_r, wnin_r, bnin_r, bnode_r, we2_r,
             wno0_r, bnout_r, weo0_r, beout_r,
             am0_r, abias_r, tconst_r, w3_r, afno_r, afeo_r):
        af_v = af_r[...]
        ai = _dot(af_v, wnin_r[...]) + bnin_r[...]
        am0_r[...] = jnp.maximum(ai, 0.0)
        abias_r[...] = ai + gbfn_r[...] + bnode_r[...]
        tconst_r[...] = _dot(gaf_r[...], we2_r[...])
        w3_r[...] = _dot(af_v, we2_r[...])
        afno_r[...] = _dot(af_v, wno0_r[...]) + bnout_r[...]
        afeo_r[...] = _dot(af_v, weo0_r[...]) + beout_r[...]

    n = NAP // BA
    f = jax.ShapeDtypeStruct((NAP, D), F32)
    return pl.pallas_call(
        body,
        grid=(n,),
        in_specs=[_row_spec(BA, D), _row_spec(BA, D), _row_spec(BA, D),
                  _rep_spec((D, D)), _rep_spec((1, D)),
                  _rep_spec((1, D)), _rep_spec((D, D)), _rep_spec((D, D)),
                  _rep_spec((1, D)), _rep_spec((D, D)), _rep_spec((1, D))],
        out_specs=[_row_spec(BA, D)] * 6,
        out_shape=[f] * 6,
    )(af, agg_af, agg_bfn, W_nin, b_nin, b_node, We2,
      Wno0, b_nout, Weo0, b_eout)


def _tc_init_bond(bf, W_ein, b_ein, b_edge, We0, Wn2):
    def body(bf_r, wein_r, bein_r, bedge_r, we0_r, wn2_r,
             bm0_r, bias2_r, u0_r, bfn_r):
        bf_v = bf_r[...]
        bi = _dot(bf_v, wein_r[...]) + bein_r[...]
        bm0 = jnp.maximum(bi, 0.0)
        bm0_r[...] = bm0
        bias2_r[...] = bi + bedge_r[...]
        u0_r[...] = _dot(bm0, we0_r[...])
        bfn_r[...] = _dot(bf_v, wn2_r[...])

    n = NB // BB
    f = jax.ShapeDtypeStruct((NB, D), F32)
    return pl.pallas_call(
        body,
        grid=(n,),
        in_specs=[_row_spec(BB, BD), _rep_spec((BD, D)), _rep_spec((1, D)),
                  _rep_spec((1, D)), _rep_spec((D, D)), _rep_spec((BD, D))],
        out_specs=[_row_spec(BB, D)] * 4,
        out_shape=[f] * 4,
    )(bf, W_ein, b_ein, b_edge, We0, Wn2)


def _tc_atom_update(abias, G1, G2, w3, Wn0, Wn1, We1):
    def body(abias_r, g1_r, g2_r, w3_r, wn0_r, wn1_r, we1_r, am_r, vp_r):
        am = jnp.maximum(
            abias_r[...] + _dot(g1_r[...], wn0_r[...])
            + _dot(g2_r[...], wn1_r[...]), 0.0)
        am_r[...] = am
        vp_r[...] = _dot(am, we1_r[...]) + w3_r[...]

    n = NAP // BA
    f = jax.ShapeDtypeStruct((NAP, D), F32)
    return pl.pallas_call(
        body,
        grid=(n,),
        in_specs=[_row_spec(BA, D)] * 4 + [_rep_spec((D, D))] * 3,
        out_specs=[_row_spec(BA, D)] * 2,
        out_shape=[f] * 2,
    )(abias, G1, G2, w3, Wn0, Wn1, We1)


def _tc_t(G2, G3, tconst, We0, We1):
    def body(g2_r, g3_r, tc_r, we0_r, we1_r, t_r):
        t_r[...] = (_dot(g2_r[...], we0_r[...]) + _dot(g3_r[...], we1_r[...])
                    + tc_r[...])

    n = NAP // BA
    return pl.pallas_call(
        body,
        grid=(n,),
        in_specs=[_row_spec(BA, D)] * 3 + [_rep_spec((D, D))] * 2,
        out_specs=_row_spec(BA, D),
        out_shape=jax.ShapeDtypeStruct((NAP, D), F32),
    )(G2, G3, tconst, We0, We1)


def _tc_bond_update(bias2, g, We0, with_u):
    def body_u(bias2_r, g_r, we0_r, bm_r, u_r):
        bm = jnp.maximum(bias2_r[...] + g_r[...], 0.0)
        bm_r[...] = bm
        u_r[...] = _dot(bm, we0_r[...])

    def body_nou(bias2_r, g_r, we0_r, bm_r):
        bm_r[...] = jnp.maximum(bias2_r[...] + g_r[...], 0.0)

    n = NB // BB
    f = jax.ShapeDtypeStruct((NB, D), F32)
    if with_u:
        return pl.pallas_call(
            body_u,
            grid=(n,),
            in_specs=[_row_spec(BB, D), _row_spec(BB, D), _rep_spec((D, D))],
            out_specs=[_row_spec(BB, D)] * 2,
            out_shape=[f] * 2,
        )(bias2, g, We0)
    return pl.pallas_call(
        body_nou,
        grid=(n,),
        in_specs=[_row_spec(BB, D), _row_spec(BB, D), _rep_spec((D, D))],
        out_specs=_row_spec(BB, D),
        out_shape=f,
    )(bias2, g, We0)


def _tc_final(afno, A, Wno1, afeo, B, Weo1):
    def body(afno_r, a_r, wno1_r, afeo_r, b_r, weo1_r, ao_r, bo_r):
        ao_r[...] = jnp.maximum(afno_r[...] + _dot(a_r[...], wno1_r[...]), 0.0)
        bo_r[...] = jnp.maximum(afeo_r[...] + _dot(b_r[...], weo1_r[...]), 0.0)

    n = NAP // BA
    f = jax.ShapeDtypeStruct((NAP, D), F32)
    return pl.pallas_call(
        body,
        grid=(n,),
        in_specs=[_row_spec(BA, D), _row_spec(BA, D), _rep_spec((D, D)),
                  _row_spec(BA, D), _row_spec(BA, D), _rep_spec((D, D))],
        out_specs=[_row_spec(BA, D)] * 2,
        out_shape=[f] * 2,
    )(afno, A, Wno1, afeo, B, Weo1)


def kernel(atom_features, bond_features, a2a, a2b, b2a, b2revb,
           W_nin, b_nin, W_ein, b_ein, W_node, b_node, W_edge, b_edge,
           W_nout, b_nout, W_eout, b_eout):
    H = D
    Wn0, Wn1, Wn2 = W_node[:H], W_node[H:2 * H], W_node[2 * H:]
    We0, We1, We2 = W_edge[:H], W_edge[H:2 * H], W_edge[2 * H:]
    Wno0, Wno1 = W_nout[:H], W_nout[H:]
    Weo0, Weo1 = W_eout[:H], W_eout[H:]
    b_nin2 = b_nin[None, :]
    b_ein2 = b_ein[None, :]
    b_node2 = b_node[None, :]
    b_edge2 = b_edge[None, :]
    b_nout2 = b_nout[None, :]
    b_eout2 = b_eout[None, :]

    af_p = jnp.pad(atom_features, ((0, NAP - NA), (0, 0)))
    a2a_f = jnp.pad(a2a, ((0, NAP - NA), (0, 0))).reshape(-1)
    a2b_f = jnp.pad(a2b, ((0, NAP - NA), (0, 0))).reshape(-1)
    rev_a = jnp.take(b2a, b2revb)
    b2a_p = jnp.pad(b2a, (0, NBP - NB))
    rev_a_p = jnp.pad(rev_a, (0, NBP - NB))
    b2revb_p = jnp.pad(b2revb, (0, NBP - NB))

    seg_s = _make_segsum_small()
    seg_b = _make_segsum_big()
    gather3 = _make_gather3()

    bm, bias2, u, bfn = _tc_init_bond(
        bond_features, W_ein, b_ein2, b_edge2, We0, Wn2)
    agg_af = seg_s(af_p, a2a_f)
    agg_bfn = seg_b(bfn, a2b_f)
    am, abias, tconst, w3, afno, afeo = _tc_init_atom(
        af_p, agg_af, agg_bfn, W_nin, b_nin2, b_node2,
        We2, Wno0, b_nout2, Weo0, b_eout2)

    G3 = None
    for step in range(DEPTH_M1):
        G1 = seg_s(am, a2a_f)
        G2 = seg_b(bm, a2b_f)
        am, vp = _tc_atom_update(abias, G1, G2, w3, Wn0, Wn1, We1)
        G3 = seg_s(am, a2a_f)
        t = _tc_t(G2, G3, tconst, We0, We1)
        g = gather3(t, vp, u, b2a_p, rev_a_p, b2revb_p)
        last = step == DEPTH_M1 - 1
        if last:
            bm = _tc_bond_update(bias2, g, We0, with_u=False)
        else:
            bm, u = _tc_bond_update(bias2, g, We0, with_u=True)

    B = seg_b(bm, a2b_f)
    atom_out, bond_out = _tc_final(afno, G3, Wno1, afeo, B, Weo1)
    return atom_out[:NA], bond_out[:NA]
